# baseline (speedup 1.0000x reference)
import functools

import jax
import jax.numpy as jnp
from jax import lax
from jax.experimental import pallas as pl
from jax.experimental.pallas import tpu as pltpu
from jax.experimental.pallas import tpu_sc as plsc

E = 16
D = 2048
H = 2048
T = 4096

M = 128
G = T // M + E
GP = G * M
HB = 1024
TB = 1024

_NC = 2
_NS = 16
_NW = _NC * _NS


def _router_body(x_ref, wr_ref, br_ref, probs_ref, counts_ref, cum_ref):
    i = pl.program_id(0)
    logits = jnp.dot(x_ref[...], wr_ref[...],
                     preferred_element_type=jnp.float32) + br_ref[...]
    m = jnp.max(logits, axis=-1, keepdims=True)
    ex = jnp.exp(logits - m)
    s = jnp.sum(ex, axis=-1, keepdims=True)
    probs = ex / s
    probs_ref[...] = probs
    routes = jnp.argmax(probs, axis=-1).astype(jnp.int32)
    iota = lax.broadcasted_iota(jnp.int32, (TB, 128), 1)
    onehot = (iota == routes[:, None]).astype(jnp.bfloat16)
    hist = jnp.sum(onehot.astype(jnp.float32), axis=0, keepdims=True)

    @pl.when(i == 0)
    def _():
        counts_ref[...] = jnp.zeros_like(counts_ref)

    carry = counts_ref[0:1, :]
    r_i = lax.broadcasted_iota(jnp.int32, (TB, TB), 0)
    c_i = lax.broadcasted_iota(jnp.int32, (TB, TB), 1)
    tri = (c_i < r_i).astype(jnp.bfloat16)
    within = jnp.dot(tri, onehot, preferred_element_type=jnp.float32)
    cum_ref[...] = within[:, :E] + carry[:, :E]
    counts_ref[0:1, :] = carry + hist


def _router(x, Wr, br):
    return pl.pallas_call(
        _router_body,
        grid=(T // TB,),
        in_specs=[
            pl.BlockSpec((TB, D), lambda i: (i, 0)),
            pl.BlockSpec((D, E), lambda i: (0, 0)),
            pl.BlockSpec((1, E), lambda i: (0, 0)),
        ],
        out_specs=[
            pl.BlockSpec((TB, E), lambda i: (i, 0)),
            pl.BlockSpec((8, 128), lambda i: (0, 0)),
            pl.BlockSpec((TB, E), lambda i: (i, 0)),
        ],
        out_shape=[
            jax.ShapeDtypeStruct((T, E), jnp.float32),
            jax.ShapeDtypeStruct((8, 128), jnp.float32),
            jax.ShapeDtypeStruct((T, E), jnp.float32),
        ],
    )(x, Wr, br.reshape(1, E))


_CH = 32


def _permute_rows(src, idx, n_out, indirect_writes):
    B = idx.shape[0]
    Dd = src.shape[1]
    b_per_w = B // _NW
    CH = _CH
    nch = b_per_w // CH
    mesh = plsc.VectorSubcoreMesh(core_axis_name="c", subcore_axis_name="s")

    @functools.partial(
        pl.kernel,
        out_type=jax.ShapeDtypeStruct((n_out, Dd), jnp.float32),
        mesh=mesh,
        scratch_types=[
            pltpu.VMEM((nch, CH), jnp.int32),
            pltpu.VMEM((CH, Dd), jnp.float32),
            pltpu.SemaphoreType.DMA,
        ],
    )
    def k(src_hbm, idx_hbm, out_hbm, idx_v, rows_v, sem):
        wid = lax.axis_index("s") * _NC + lax.axis_index("c")
        base = wid * b_per_w
        pltpu.sync_copy(idx_hbm.at[wid], idx_v)
        for j in range(nch):
            if indirect_writes:
                pltpu.sync_copy(src_hbm.at[pl.ds(base + j * CH, CH)], rows_v)
                pltpu.async_copy(rows_v, out_hbm.at[idx_v.at[j]], sem).wait()
            else:
                pltpu.async_copy(src_hbm.at[idx_v.at[j]], rows_v, sem).wait()
                pltpu.sync_copy(rows_v, out_hbm.at[pl.ds(base + j * CH, CH)])

    return k(src, idx.reshape(_NW, nch, CH))


def _gather_rows(table, idx):
    return _permute_rows(table, idx, idx.shape[0], indirect_writes=False)


def _scatter_rows(src, pos, n_out):
    return _permute_rows(src, pos, n_out, indirect_writes=True)


def _index_body(probs_ref, cum_ref, counts_ref, pos_ref, te_ref, xsidx_ref,
                rn_ref, fr_ref, nxe_ref):
    probs = probs_ref[...]
    routes = jnp.argmax(probs, axis=-1).astype(jnp.int32)
    lane = lax.broadcasted_iota(jnp.int32, (T, E), 1)
    onehot = (lane == routes[:, None]).astype(jnp.float32)
    rank = jnp.sum(cum_ref[...] * onehot, axis=-1, keepdims=True)

    counts = counts_ref[0:1, :]
    padded = jnp.ceil(counts * (1.0 / M)) * M
    r_i = lax.broadcasted_iota(jnp.int32, (128, 128), 0)
    c_i = lax.broadcasted_iota(jnp.int32, (128, 128), 1)
    upper = (r_i < c_i).astype(jnp.bfloat16)
    poffs = jnp.dot(padded.astype(jnp.bfloat16), upper,
                    preferred_element_type=jnp.float32)

    poff_tok = jnp.sum(poffs[:, :E] * onehot, axis=-1, keepdims=True)
    pos_ref[...] = (poff_tok + rank).astype(jnp.int32)

    g_i = (lax.broadcasted_iota(jnp.int32, (G, 128), 0) * M).astype(jnp.float32)
    lane_g = lax.broadcasted_iota(jnp.int32, (G, 128), 1)
    hit = jnp.logical_and(lane_g < E, poffs <= g_i)
    te_ref[...] = (jnp.sum(hit.astype(jnp.float32), axis=-1, keepdims=True)
                   ).astype(jnp.int32) - 1

    n_used = jnp.sum(padded[0:1, :E] * (1.0 / M), axis=-1, keepdims=True)
    g_col = lax.broadcasted_iota(jnp.int32, (G, 1), 0)
    xsidx_ref[...] = jnp.minimum(g_col, n_used.astype(jnp.int32) - 1)

    nonempty = jnp.logical_and(lane_g < E, (padded > 0.0))
    rn_f = jnp.sum(jnp.logical_and(nonempty, poffs <= g_i).astype(jnp.float32),
                   axis=-1, keepdims=True) - 1.0
    rn_ref[...] = rn_f.astype(jnp.int32)

    te_mask = (lane_g == te_ref[...]).astype(jnp.float32)
    poth = jnp.sum(poffs * te_mask, axis=-1, keepdims=True)
    fr_ref[...] = (poth == (g_col * M).astype(jnp.float32)).astype(jnp.int32)

    rk = jnp.dot(nonempty[0:1, :].astype(jnp.bfloat16), upper,
                 preferred_element_type=jnp.float32)
    hit_n = jnp.logical_and(nonempty, rk == (rn_f + 1.0))
    s_n = jnp.sum(hit_n.astype(jnp.float32), axis=-1, keepdims=True)
    lane_f = lane_g.astype(jnp.float32)
    nxe_f = (jnp.sum(lane_f * hit_n.astype(jnp.float32), axis=-1, keepdims=True)
             + te_ref[...].astype(jnp.float32) * (1.0 - s_n))
    nxe_ref[...] = nxe_f.astype(jnp.int32)


def _index_kernel(probs, counts8, cum):
    return pl.pallas_call(
        _index_body,
        in_specs=[
            pl.BlockSpec((T, E), lambda: (0, 0)),
            pl.BlockSpec((T, E), lambda: (0, 0)),
            pl.BlockSpec((8, 128), lambda: (0, 0)),
        ],
        out_specs=[
            pl.BlockSpec((T, 1), lambda: (0, 0)),
        ] + [pl.BlockSpec((G, 1), lambda: (0, 0))] * 5,
        out_shape=[
            jax.ShapeDtypeStruct((T, 1), jnp.int32),
        ] + [jax.ShapeDtypeStruct((G, 1), jnp.int32)] * 5,
    )(probs, cum, counts8)


def _mm_body(te_ref, xi_ref, rn_ref, fr_ref, nxe_ref,
             xs_ref, we_hbm, be_ref, out_ref, we_buf, sems):
    g = pl.program_id(0)
    r = rn_ref[g]
    slot = lax.rem(r, 2)

    def we_copy(e, s):
        return pltpu.make_async_copy(we_hbm.at[e], we_buf.at[s], sems.at[s])

    @pl.when(g == 0)
    def _():
        we_copy(te_ref[0], 0).start()

    @pl.when((fr_ref[g] == 1) & (nxe_ref[g] != te_ref[g]))
    def _():
        we_copy(nxe_ref[g], lax.rem(r + 1, 2)).start()

    @pl.when(fr_ref[g] == 1)
    def _():
        we_copy(te_ref[g], slot).wait()

    @pl.when(xi_ref[g] == g)
    def _():
        out_ref[...] = jnp.dot(xs_ref[...].astype(jnp.bfloat16),
                               we_buf[slot].astype(jnp.bfloat16),
                               preferred_element_type=jnp.float32) + be_ref[0]


def _grouped_matmul(te, xsidx, rn, fr, nxe, xs, We, be):
    grid_spec = pltpu.PrefetchScalarGridSpec(
        num_scalar_prefetch=5,
        grid=(G,),
        in_specs=[
            pl.BlockSpec((M, D), lambda g, *sc: (sc[1][g], 0)),
            pl.BlockSpec(memory_space=pl.ANY),
            pl.BlockSpec((1, 1, H), lambda g, *sc: (sc[0][g], 0, 0)),
        ],
        out_specs=pl.BlockSpec((M, H), lambda g, *sc: (sc[1][g], 0)),
        scratch_shapes=[
            pltpu.VMEM((2, D, H), jnp.float32),
            pltpu.SemaphoreType.DMA((2,)),
        ],
    )
    return pl.pallas_call(
        _mm_body,
        grid_spec=grid_spec,
        out_shape=jax.ShapeDtypeStruct((GP, H), jnp.float32),
    )(te, xsidx, rn, fr, nxe, xs, We, be.reshape(E, 1, H))


def kernel(x, Wr, br, We, be):
    probs, counts8, cum = _router(x, Wr, br)
    counts = counts8[0, :E]

    pos2, te2, xsidx2, rn2, fr2, nxe2 = _index_kernel(probs, counts8, cum)
    pos = pos2.reshape(T)
    te = te2.reshape(G)
    xsidx = xsidx2.reshape(G)
    rn = rn2.reshape(G)
    fr = fr2.reshape(G)
    nxe = nxe2.reshape(G)

    xs = _scatter_rows(x, pos, GP)
    ys = _grouped_matmul(te, xsidx, rn, fr, nxe, xs, We, be)
    out = _gather_rows(ys, pos)

    return out, probs, counts


# speedup vs baseline: 1.0022x; 1.0022x over previous
import functools

import jax
import jax.numpy as jnp
from jax import lax
from jax.experimental import pallas as pl
from jax.experimental.pallas import tpu as pltpu
from jax.experimental.pallas import tpu_sc as plsc

E = 16
D = 2048
H = 2048
T = 4096

M = 128
G = T // M + E
GP = G * M
HB = 1024
TB = 512

_NC = 2
_NS = 16
_NW = _NC * _NS


def _router_body(x_ref, wr_ref, br_ref, probs_ref, counts_ref, cum_ref):
    i = pl.program_id(0)
    logits = jnp.dot(x_ref[...], wr_ref[...],
                     preferred_element_type=jnp.float32) + br_ref[...]
    m = jnp.max(logits, axis=-1, keepdims=True)
    ex = jnp.exp(logits - m)
    s = jnp.sum(ex, axis=-1, keepdims=True)
    probs = ex / s
    probs_ref[...] = probs
    routes = jnp.argmax(probs, axis=-1).astype(jnp.int32)
    iota = lax.broadcasted_iota(jnp.int32, (TB, 128), 1)
    onehot = (iota == routes[:, None]).astype(jnp.bfloat16)
    hist = jnp.sum(onehot.astype(jnp.float32), axis=0, keepdims=True)

    @pl.when(i == 0)
    def _():
        counts_ref[...] = jnp.zeros_like(counts_ref)

    carry = counts_ref[0:1, :]
    r_i = lax.broadcasted_iota(jnp.int32, (TB, TB), 0)
    c_i = lax.broadcasted_iota(jnp.int32, (TB, TB), 1)
    tri = (c_i < r_i).astype(jnp.bfloat16)
    within = jnp.dot(tri, onehot, preferred_element_type=jnp.float32)
    cum_ref[...] = within[:, :E] + carry[:, :E]
    counts_ref[0:1, :] = carry + hist


def _router(x, Wr, br):
    return pl.pallas_call(
        _router_body,
        grid=(T // TB,),
        in_specs=[
            pl.BlockSpec((TB, D), lambda i: (i, 0)),
            pl.BlockSpec((D, E), lambda i: (0, 0)),
            pl.BlockSpec((1, E), lambda i: (0, 0)),
        ],
        out_specs=[
            pl.BlockSpec((TB, E), lambda i: (i, 0)),
            pl.BlockSpec((8, 128), lambda i: (0, 0)),
            pl.BlockSpec((TB, E), lambda i: (i, 0)),
        ],
        out_shape=[
            jax.ShapeDtypeStruct((T, E), jnp.float32),
            jax.ShapeDtypeStruct((8, 128), jnp.float32),
            jax.ShapeDtypeStruct((T, E), jnp.float32),
        ],
    )(x, Wr, br.reshape(1, E))


_CH = 32


def _permute_rows(src, idx, n_out, indirect_writes):
    B = idx.shape[0]
    Dd = src.shape[1]
    b_per_w = B // _NW
    CH = _CH
    nch = b_per_w // CH
    mesh = plsc.VectorSubcoreMesh(core_axis_name="c", subcore_axis_name="s")

    @functools.partial(
        pl.kernel,
        out_type=jax.ShapeDtypeStruct((n_out, Dd), jnp.float32),
        mesh=mesh,
        scratch_types=[
            pltpu.VMEM((nch, CH), jnp.int32),
            pltpu.VMEM((CH, Dd), jnp.float32),
            pltpu.SemaphoreType.DMA,
        ],
    )
    def k(src_hbm, idx_hbm, out_hbm, idx_v, rows_v, sem):
        wid = lax.axis_index("s") * _NC + lax.axis_index("c")
        base = wid * b_per_w
        pltpu.sync_copy(idx_hbm.at[wid], idx_v)
        for j in range(nch):
            if indirect_writes:
                pltpu.sync_copy(src_hbm.at[pl.ds(base + j * CH, CH)], rows_v)
                pltpu.async_copy(rows_v, out_hbm.at[idx_v.at[j]], sem).wait()
            else:
                pltpu.async_copy(src_hbm.at[idx_v.at[j]], rows_v, sem).wait()
                pltpu.sync_copy(rows_v, out_hbm.at[pl.ds(base + j * CH, CH)])

    return k(src, idx.reshape(_NW, nch, CH))


def _gather_rows(table, idx):
    return _permute_rows(table, idx, idx.shape[0], indirect_writes=False)


def _scatter_rows(src, pos, n_out):
    return _permute_rows(src, pos, n_out, indirect_writes=True)


def _index_body(probs_ref, cum_ref, counts_ref, pos_ref, te_ref, xsidx_ref,
                rn_ref, fr_ref, nxe_ref):
    probs = probs_ref[...]
    routes = jnp.argmax(probs, axis=-1).astype(jnp.int32)
    lane = lax.broadcasted_iota(jnp.int32, (T, E), 1)
    onehot = (lane == routes[:, None]).astype(jnp.float32)
    rank = jnp.sum(cum_ref[...] * onehot, axis=-1, keepdims=True)

    counts = counts_ref[0:1, :]
    padded = jnp.ceil(counts * (1.0 / M)) * M
    r_i = lax.broadcasted_iota(jnp.int32, (128, 128), 0)
    c_i = lax.broadcasted_iota(jnp.int32, (128, 128), 1)
    upper = (r_i < c_i).astype(jnp.bfloat16)
    poffs = jnp.dot(padded.astype(jnp.bfloat16), upper,
                    preferred_element_type=jnp.float32)

    poff_tok = jnp.sum(poffs[:, :E] * onehot, axis=-1, keepdims=True)
    pos_ref[...] = (poff_tok + rank).astype(jnp.int32)

    g_i = (lax.broadcasted_iota(jnp.int32, (G, 128), 0) * M).astype(jnp.float32)
    lane_g = lax.broadcasted_iota(jnp.int32, (G, 128), 1)
    hit = jnp.logical_and(lane_g < E, poffs <= g_i)
    te_ref[...] = (jnp.sum(hit.astype(jnp.float32), axis=-1, keepdims=True)
                   ).astype(jnp.int32) - 1

    n_used = jnp.sum(padded[0:1, :E] * (1.0 / M), axis=-1, keepdims=True)
    g_col = lax.broadcasted_iota(jnp.int32, (G, 1), 0)
    xsidx_ref[...] = jnp.minimum(g_col, n_used.astype(jnp.int32) - 1)

    nonempty = jnp.logical_and(lane_g < E, (padded > 0.0))
    rn_f = jnp.sum(jnp.logical_and(nonempty, poffs <= g_i).astype(jnp.float32),
                   axis=-1, keepdims=True) - 1.0
    rn_ref[...] = rn_f.astype(jnp.int32)

    te_mask = (lane_g == te_ref[...]).astype(jnp.float32)
    poth = jnp.sum(poffs * te_mask, axis=-1, keepdims=True)
    fr_ref[...] = (poth == (g_col * M).astype(jnp.float32)).astype(jnp.int32)

    rk = jnp.dot(nonempty[0:1, :].astype(jnp.bfloat16), upper,
                 preferred_element_type=jnp.float32)
    hit_n = jnp.logical_and(nonempty, rk == (rn_f + 1.0))
    s_n = jnp.sum(hit_n.astype(jnp.float32), axis=-1, keepdims=True)
    lane_f = lane_g.astype(jnp.float32)
    nxe_f = (jnp.sum(lane_f * hit_n.astype(jnp.float32), axis=-1, keepdims=True)
             + te_ref[...].astype(jnp.float32) * (1.0 - s_n))
    nxe_ref[...] = nxe_f.astype(jnp.int32)


def _index_kernel(probs, counts8, cum):
    return pl.pallas_call(
        _index_body,
        in_specs=[
            pl.BlockSpec((T, E), lambda: (0, 0)),
            pl.BlockSpec((T, E), lambda: (0, 0)),
            pl.BlockSpec((8, 128), lambda: (0, 0)),
        ],
        out_specs=[
            pl.BlockSpec((T, 1), lambda: (0, 0)),
        ] + [pl.BlockSpec((G, 1), lambda: (0, 0))] * 5,
        out_shape=[
            jax.ShapeDtypeStruct((T, 1), jnp.int32),
        ] + [jax.ShapeDtypeStruct((G, 1), jnp.int32)] * 5,
    )(probs, cum, counts8)


def _mm_body(te_ref, xi_ref, rn_ref, fr_ref, nxe_ref,
             xs_ref, we_hbm, be_ref, out_ref, we_buf, sems):
    g = pl.program_id(0)
    r = rn_ref[g]
    slot = lax.rem(r, 2)

    def we_copy(e, s):
        return pltpu.make_async_copy(we_hbm.at[e], we_buf.at[s], sems.at[s])

    @pl.when(g == 0)
    def _():
        we_copy(te_ref[0], 0).start()

    @pl.when((fr_ref[g] == 1) & (nxe_ref[g] != te_ref[g]))
    def _():
        we_copy(nxe_ref[g], lax.rem(r + 1, 2)).start()

    @pl.when(fr_ref[g] == 1)
    def _():
        we_copy(te_ref[g], slot).wait()

    @pl.when(xi_ref[g] == g)
    def _():
        out_ref[...] = jnp.dot(xs_ref[...].astype(jnp.bfloat16),
                               we_buf[slot].astype(jnp.bfloat16),
                               preferred_element_type=jnp.float32) + be_ref[0]


def _grouped_matmul(te, xsidx, rn, fr, nxe, xs, We, be):
    grid_spec = pltpu.PrefetchScalarGridSpec(
        num_scalar_prefetch=5,
        grid=(G,),
        in_specs=[
            pl.BlockSpec((M, D), lambda g, *sc: (sc[1][g], 0)),
            pl.BlockSpec(memory_space=pl.ANY),
            pl.BlockSpec((1, 1, H), lambda g, *sc: (sc[0][g], 0, 0)),
        ],
        out_specs=pl.BlockSpec((M, H), lambda g, *sc: (sc[1][g], 0)),
        scratch_shapes=[
            pltpu.VMEM((2, D, H), jnp.float32),
            pltpu.SemaphoreType.DMA((2,)),
        ],
    )
    return pl.pallas_call(
        _mm_body,
        grid_spec=grid_spec,
        out_shape=jax.ShapeDtypeStruct((GP, H), jnp.float32),
    )(te, xsidx, rn, fr, nxe, xs, We, be.reshape(E, 1, H))


def kernel(x, Wr, br, We, be):
    probs, counts8, cum = _router(x, Wr, br)
    counts = counts8[0, :E]

    pos2, te2, xsidx2, rn2, fr2, nxe2 = _index_kernel(probs, counts8, cum)
    pos = pos2.reshape(T)
    te = te2.reshape(G)
    xsidx = xsidx2.reshape(G)
    rn = rn2.reshape(G)
    fr = fr2.reshape(G)
    nxe = nxe2.reshape(G)

    xs = _scatter_rows(x, pos, GP)
    ys = _grouped_matmul(te, xsidx, rn, fr, nxe, xs, We, be)
    out = _gather_rows(ys, pos)

    return out, probs, counts
